# baseline (speedup 1.0000x reference)
import jax
import jax.numpy as jnp
from jax.experimental import pallas as pl
from jax.experimental.pallas import tpu as pltpu

_BR = 8
_C = 1024


def _body(x_ref, out_ref):
    i = pl.program_id(0)
    seq = x_ref.shape[1]
    nch = seq // _C

    cnt = jnp.zeros((_BR, _C), jnp.float32)
    for c in range(nch):
        xt = x_ref[:, c * _C:(c + 1) * _C]
        cnt = cnt + jnp.where(xt != 0.0, 1.0, 0.0)
    real_len = jnp.sum(cnt, axis=1, keepdims=True)

    iota = jax.lax.broadcasted_iota(
        jnp.int32, (_BR, _C), 1).astype(jnp.float32)
    acc = jnp.zeros((_BR, _C), jnp.float32)
    prev_tail = x_ref[:, 0:1]
    for c in range(nch):
        xt = x_ref[:, c * _C:(c + 1) * _C]
        shifted = jnp.concatenate([prev_tail, xt[:, :_C - 1]], axis=1)
        d = jnp.abs(xt - shifted)
        thresh = real_len - float(c * _C)
        acc = acc + jnp.where(iota < thresh, d, 0.0)
        prev_tail = xt[:, _C - 1:_C]
    rowsum = jnp.sum(acc, axis=1, keepdims=True)

    per_row = rowsum / real_len
    row_id = (jax.lax.broadcasted_iota(jnp.int32, (_BR, 1), 0)
              + i * _BR).astype(jnp.float32)
    per_row = jnp.where(row_id >= 1.0, per_row, 0.0)
    tot = jnp.sum(per_row, axis=0, keepdims=True)
    out_ref[...] = jnp.broadcast_to(tot[None], (1, 1, 128))


def _consecutive_loss(x):
    bsz, seq = x.shape
    nb = bsz // _BR
    partials = pl.pallas_call(
        _body,
        grid=(nb,),
        in_specs=[pl.BlockSpec((_BR, seq), lambda i: (i, 0))],
        out_specs=pl.BlockSpec((1, 1, 128), lambda i: (i, 0, 0)),
        out_shape=jax.ShapeDtypeStruct((nb, 1, 128), jnp.float32),
        compiler_params=pltpu.CompilerParams(
            dimension_semantics=("parallel",),
        ),
    )(x)
    return jnp.sum(partials[:, 0, 0]) / bsz


def kernel(x):
    return _consecutive_loss(x)


# speedup vs baseline: 3.2570x; 3.2570x over previous
import jax
import jax.numpy as jnp
from jax.experimental import pallas as pl
from jax.experimental.pallas import tpu as pltpu

_BR = 64
_TILES = _BR // 8
_C = 1024


def _body(x_ref, out_ref):
    i = pl.program_id(0)
    seq = x_ref.shape[1]
    nch = seq // _C
    iota = jax.lax.broadcasted_iota(
        jnp.int32, (8, _C), 1).astype(jnp.float32)

    lens = []
    for t in range(_TILES):
        cnt = jnp.zeros((8, _C), jnp.float32)
        for c in range(nch):
            xt = x_ref[t * 8:(t + 1) * 8, c * _C:(c + 1) * _C]
            cnt = cnt + jnp.where(xt != 0.0, 1.0, 0.0)
        lens.append(jnp.sum(cnt, axis=1, keepdims=True))

    tile_tots = []
    for t in range(_TILES):
        real_len = lens[t]
        acc = jnp.zeros((8, _C), jnp.float32)
        prev_tail = x_ref[t * 8:(t + 1) * 8, 0:1]
        for c in range(nch):
            xt = x_ref[t * 8:(t + 1) * 8, c * _C:(c + 1) * _C]
            shifted = jnp.concatenate([prev_tail, xt[:, :_C - 1]], axis=1)
            d = jnp.abs(xt - shifted)
            acc = acc + jnp.where(iota < (real_len - float(c * _C)), d, 0.0)
            prev_tail = xt[:, _C - 1:_C]
        rowsum = jnp.sum(acc, axis=1, keepdims=True)
        per_row = rowsum / real_len
        row_id = (jax.lax.broadcasted_iota(jnp.int32, (8, 1), 0)
                  + (i * _BR + t * 8)).astype(jnp.float32)
        tile_tots.append(jnp.where(row_id >= 1.0, per_row, 0.0))

    tot = tile_tots[0]
    for t in range(1, _TILES):
        tot = tot + tile_tots[t]
    tot = jnp.sum(tot, axis=0, keepdims=True)
    out_ref[...] = jnp.broadcast_to(tot[None], (1, 1, 128))


def _consecutive_loss(x):
    bsz, seq = x.shape
    nb = bsz // _BR
    partials = pl.pallas_call(
        _body,
        grid=(nb,),
        in_specs=[pl.BlockSpec((_BR, seq), lambda i: (i, 0))],
        out_specs=pl.BlockSpec((1, 1, 128), lambda i: (i, 0, 0)),
        out_shape=jax.ShapeDtypeStruct((nb, 1, 128), jnp.float32),
        compiler_params=pltpu.CompilerParams(
            dimension_semantics=("parallel",),
        ),
    )(x)
    return jnp.sum(partials[:, 0, 0]) / bsz


def kernel(x):
    return _consecutive_loss(x)
